# baseline (speedup 1.0000x reference)
import jax
import jax.numpy as jnp
from jax.experimental import pallas as pl
from jax.experimental.pallas import tpu as pltpu


def _copy_body(node_in, edge_in, node_out, edge_out, sem_n, sem_e):
    ncopy = pltpu.make_async_copy(node_in, node_out, sem_n)
    ecopy = pltpu.make_async_copy(edge_in, edge_out, sem_e)
    ncopy.start()
    ecopy.start()
    ncopy.wait()
    ecopy.wait()


def kernel(node_latents_from, node_latents_to, edge_latents, edge_index, receivers_count):
    new_node_latents, new_edge_latents = pl.pallas_call(
        _copy_body,
        out_shape=(
            jax.ShapeDtypeStruct(node_latents_to.shape, node_latents_to.dtype),
            jax.ShapeDtypeStruct(edge_latents.shape, edge_latents.dtype),
        ),
        in_specs=(
            pl.BlockSpec(memory_space=pl.ANY),
            pl.BlockSpec(memory_space=pl.ANY),
        ),
        out_specs=(
            pl.BlockSpec(memory_space=pl.ANY),
            pl.BlockSpec(memory_space=pl.ANY),
        ),
        scratch_shapes=(
            pltpu.SemaphoreType.DMA,
            pltpu.SemaphoreType.DMA,
        ),
    )(node_latents_to, edge_latents)
    return (new_node_latents, new_edge_latents)


# speedup vs baseline: 17.4047x; 17.4047x over previous
import jax
import jax.numpy as jnp
from jax.experimental import pallas as pl


def _copy_body(node_in, edge_in, node_out, edge_out):
    node_out[...] = node_in[...]
    edge_out[...] = edge_in[...]


def _pick_grid(n_node_rows, n_edge_rows, target):
    for g in range(target, 0, -1):
        if n_node_rows % g == 0 and n_edge_rows % g == 0:
            return g
    return 1


def kernel(node_latents_from, node_latents_to, edge_latents, edge_index, receivers_count):
    n_nodes, d_feat = node_latents_to.shape
    n_edges, d_edge = edge_latents.shape

    row = 128
    if (n_edges * d_edge) % row == 0:
        edge_rows = (n_edges * d_edge) // row
        edge_view = edge_latents.reshape(edge_rows, row)
    else:
        edge_rows, row = n_edges, d_edge
        edge_view = edge_latents

    g = _pick_grid(n_nodes, edge_rows, 10)
    nb, eb = n_nodes // g, edge_rows // g

    node_copy, edge_copy = pl.pallas_call(
        _copy_body,
        grid=(g,),
        in_specs=(
            pl.BlockSpec((nb, d_feat), lambda i: (i, 0)),
            pl.BlockSpec((eb, row), lambda i: (i, 0)),
        ),
        out_specs=(
            pl.BlockSpec((nb, d_feat), lambda i: (i, 0)),
            pl.BlockSpec((eb, row), lambda i: (i, 0)),
        ),
        out_shape=(
            jax.ShapeDtypeStruct((n_nodes, d_feat), node_latents_to.dtype),
            jax.ShapeDtypeStruct((edge_rows, row), edge_latents.dtype),
        ),
    )(node_latents_to, edge_view)
    return (node_copy, edge_copy.reshape(n_edges, d_edge))


# speedup vs baseline: 19.2978x; 1.1088x over previous
import jax
import jax.numpy as jnp
from jax.experimental import pallas as pl


def _copy_body(node_in, edge_in, node_out, edge_out):
    node_out[...] = node_in[...]
    edge_out[...] = edge_in[...]


def _pick_grid(n_node_rows, n_edge_rows, target):
    for g in range(target, 0, -1):
        if (n_node_rows % g == 0 and n_edge_rows % g == 0
                and (n_node_rows // g) % 8 == 0 and (n_edge_rows // g) % 8 == 0):
            return g
    return 1


def kernel(node_latents_from, node_latents_to, edge_latents, edge_index, receivers_count):
    n_nodes, d_feat = node_latents_to.shape
    n_edges, d_edge = edge_latents.shape

    row = d_edge
    edge_rows = n_edges
    edge_view = edge_latents

    g = _pick_grid(n_nodes, edge_rows, 40)
    nb, eb = n_nodes // g, edge_rows // g

    node_copy, edge_copy = pl.pallas_call(
        _copy_body,
        grid=(g,),
        in_specs=(
            pl.BlockSpec((nb, d_feat), lambda i: (i, 0)),
            pl.BlockSpec((eb, row), lambda i: (i, 0)),
        ),
        out_specs=(
            pl.BlockSpec((nb, d_feat), lambda i: (i, 0)),
            pl.BlockSpec((eb, row), lambda i: (i, 0)),
        ),
        out_shape=(
            jax.ShapeDtypeStruct((n_nodes, d_feat), node_latents_to.dtype),
            jax.ShapeDtypeStruct((edge_rows, row), edge_latents.dtype),
        ),
    )(node_latents_to, edge_view)
    return (node_copy, edge_copy)
